# Initial kernel scaffold; baseline (speedup 1.0000x reference)
import jax
import jax.numpy as jnp
from jax.experimental import pallas as pl


def kernel(x_c, pos_c, batch_c, res, pos, batch, W, b):
    raise NotImplementedError("write your pallas kernel here")


# speedup vs baseline: 3.3413x; 3.3413x over previous
import functools

import jax
import jax.numpy as jnp
from jax import lax
from jax.experimental import pallas as pl

K = 3
N_C = 4096
N_F = 16384
D_FEAT = 256
D_RES = 256
D_OUT = 512
BF = 256


def _upsample_block(pos_ref, res_ref, pos_cT_ref, pos_c_ref, x_c_ref,
                    WxT_ref, WrT_ref, b_ref, out_ref):
    posb = pos_ref[...]
    pcT = pos_cT_ref[...]
    xy = jnp.dot(posb, pcT, precision=lax.Precision.DEFAULT)
    y2 = jnp.sum(posb * posb, axis=1, keepdims=True)
    x2 = jnp.sum(pcT * pcT, axis=0, keepdims=True)
    dist = (y2 + x2) - 2.0 * xy
    iota = lax.broadcasted_iota(jnp.int32, (BF, N_C), 1)

    A = jnp.zeros((BF, N_C), dtype=jnp.float32)
    wsum = jnp.zeros((BF, 1), dtype=jnp.float32)
    for _ in range(K):
        m = jnp.min(dist, axis=1, keepdims=True)
        cand = jnp.where(dist == m, iota, N_C)
        amin = jnp.min(cand, axis=1, keepdims=True)
        onehot = (iota == amin)
        onef = onehot.astype(jnp.float32)
        psel = jnp.dot(onef, pos_c_ref[...],
                       precision=lax.Precision.HIGHEST)
        diff = psel - posb
        dexact = jnp.sum(diff * diff, axis=1, keepdims=True)
        w = 1.0 / jnp.maximum(dexact, 1e-16)
        A = A + onef * w
        wsum = wsum + w
        dist = jnp.where(onehot, jnp.float32(3.4e38), dist)

    num = jnp.dot(A, x_c_ref[...], precision=lax.Precision.HIGHEST)
    x = num / wsum
    out = (jnp.dot(x, WxT_ref[...], precision=lax.Precision.HIGHEST)
           + jnp.dot(res_ref[...], WrT_ref[...], precision=lax.Precision.HIGHEST)
           + b_ref[...])
    out_ref[...] = out


@jax.jit
def _upsample(pos_pad, res, pos_cT, pos_c_pad, x_c, WxT, WrT, b2):
    grid = (N_F // BF,)
    return pl.pallas_call(
        _upsample_block,
        grid=grid,
        in_specs=[
            pl.BlockSpec((BF, 8), lambda i: (i, 0)),
            pl.BlockSpec((BF, D_RES), lambda i: (i, 0)),
            pl.BlockSpec((8, N_C), lambda i: (0, 0)),
            pl.BlockSpec((N_C, 8), lambda i: (0, 0)),
            pl.BlockSpec((N_C, D_FEAT), lambda i: (0, 0)),
            pl.BlockSpec((D_FEAT, D_OUT), lambda i: (0, 0)),
            pl.BlockSpec((D_RES, D_OUT), lambda i: (0, 0)),
            pl.BlockSpec((1, D_OUT), lambda i: (0, 0)),
        ],
        out_specs=pl.BlockSpec((BF, D_OUT), lambda i: (i, 0)),
        out_shape=jax.ShapeDtypeStruct((N_F, D_OUT), jnp.float32),
    )(pos_pad, res, pos_cT, pos_c_pad, x_c, WxT, WrT, b2)


def kernel(x_c, pos_c, batch_c, res, pos, batch, W, b):
    pos_pad = jnp.pad(pos, ((0, 0), (0, 5)))
    pos_c_pad = jnp.pad(pos_c, ((0, 0), (0, 5)))
    pos_cT = pos_c_pad.T
    WxT = W[:, :D_FEAT].T
    WrT = W[:, D_FEAT:].T
    b2 = b.reshape(1, D_OUT)
    out = _upsample(pos_pad, res, pos_cT, pos_c_pad, x_c, WxT, WrT, b2)
    return (out, pos, batch)


# speedup vs baseline: 4.8784x; 1.4601x over previous
import functools

import jax
import jax.numpy as jnp
from jax import lax
from jax.experimental import pallas as pl

K = 3
N_C = 4096
N_F = 16384
D_FEAT = 256
D_RES = 256
D_OUT = 512
BF = 256


def _upsample_block(pos_ref, res_ref, pos_cT_ref, pos_c_ref, x_c_ref,
                    WxT_ref, WrT_ref, b_ref, out_ref):
    posb = pos_ref[...]
    pcT = pos_cT_ref[...]
    xy = jnp.dot(posb, pcT, precision=lax.Precision.DEFAULT)
    y2 = jnp.sum(posb * posb, axis=1, keepdims=True)
    x2 = jnp.sum(pcT * pcT, axis=0, keepdims=True)
    dist = (y2 + x2) - 2.0 * xy
    iota = lax.broadcasted_iota(jnp.int32, (BF, N_C), 1)

    A = jnp.zeros((BF, N_C), dtype=jnp.float32)
    wsum = jnp.zeros((BF, 1), dtype=jnp.float32)
    for _ in range(K):
        m = jnp.min(dist, axis=1, keepdims=True)
        cand = jnp.where(dist == m, iota, N_C)
        amin = jnp.min(cand, axis=1, keepdims=True)
        onehot = (iota == amin)
        onef = onehot.astype(jnp.float32)
        psel = jnp.dot(onef, pos_c_ref[...],
                       precision=lax.Precision.HIGHEST)
        diff = psel - posb
        dexact = jnp.sum(diff * diff, axis=1, keepdims=True)
        w = 1.0 / jnp.maximum(dexact, 1e-16)
        A = A + onef * w
        wsum = wsum + w
        dist = jnp.where(onehot, jnp.float32(3.4e38), dist)

    num = jnp.dot(A, x_c_ref[...], precision=lax.Precision.DEFAULT)
    x = num / wsum
    out = (jnp.dot(x, WxT_ref[...], precision=lax.Precision.DEFAULT)
           + jnp.dot(res_ref[...], WrT_ref[...], precision=lax.Precision.DEFAULT)
           + b_ref[...])
    out_ref[...] = out


@jax.jit
def _upsample(pos_pad, res, pos_cT, pos_c_pad, x_c, WxT, WrT, b2):
    grid = (N_F // BF,)
    return pl.pallas_call(
        _upsample_block,
        grid=grid,
        in_specs=[
            pl.BlockSpec((BF, 8), lambda i: (i, 0)),
            pl.BlockSpec((BF, D_RES), lambda i: (i, 0)),
            pl.BlockSpec((8, N_C), lambda i: (0, 0)),
            pl.BlockSpec((N_C, 8), lambda i: (0, 0)),
            pl.BlockSpec((N_C, D_FEAT), lambda i: (0, 0)),
            pl.BlockSpec((D_FEAT, D_OUT), lambda i: (0, 0)),
            pl.BlockSpec((D_RES, D_OUT), lambda i: (0, 0)),
            pl.BlockSpec((1, D_OUT), lambda i: (0, 0)),
        ],
        out_specs=pl.BlockSpec((BF, D_OUT), lambda i: (i, 0)),
        out_shape=jax.ShapeDtypeStruct((N_F, D_OUT), jnp.float32),
    )(pos_pad, res, pos_cT, pos_c_pad, x_c, WxT, WrT, b2)


def kernel(x_c, pos_c, batch_c, res, pos, batch, W, b):
    pos_pad = jnp.pad(pos, ((0, 0), (0, 5)))
    pos_c_pad = jnp.pad(pos_c, ((0, 0), (0, 5)))
    pos_cT = pos_c_pad.T
    WxT = W[:, :D_FEAT].T
    WrT = W[:, D_FEAT:].T
    b2 = b.reshape(1, D_OUT)
    out = _upsample(pos_pad, res, pos_cT, pos_c_pad, x_c, WxT, WrT, b2)
    return (out, pos, batch)


# speedup vs baseline: 10.8944x; 2.2332x over previous
import functools

import jax
import jax.numpy as jnp
from jax import lax
from jax.experimental import pallas as pl

K = 3
N_C = 4096
N_F = 16384
D_FEAT = 256
D_RES = 256
D_OUT = 512
BF = 256
_BIG = 3.4e38


def _upsample_block(pos_ref, res_ref, pos_cT_ref, x_c_ref,
                    WxT_ref, WrT_ref, b_ref, out_ref):
    posb = pos_ref[...]
    pcT = pos_cT_ref[...]
    xy = jnp.dot(posb, pcT, precision=lax.Precision.DEFAULT)
    y2 = jnp.sum(posb * posb, axis=1, keepdims=True)
    x2 = jnp.sum(pcT * pcT, axis=0, keepdims=True)
    dist = (y2 + x2) - 2.0 * xy

    dex = ((posb[:, 0:1] - pcT[0:1, :]) ** 2
           + (posb[:, 1:2] - pcT[1:2, :]) ** 2
           + (posb[:, 2:3] - pcT[2:3, :]) ** 2)

    iota = lax.broadcasted_iota(jnp.int32, (BF, N_C), 1)
    for _ in range(K):
        m = jnp.min(dist, axis=1, keepdims=True)
        cand = jnp.where(dist == m, iota, N_C)
        amin = jnp.min(cand, axis=1, keepdims=True)
        dist = jnp.where(iota == amin, _BIG, dist)

    sel = dist >= 3.0e38
    A = jnp.where(sel, 1.0 / jnp.maximum(dex, 1e-16), 0.0)
    wsum = jnp.sum(A, axis=1, keepdims=True)
    num = jnp.dot(A, x_c_ref[...], precision=lax.Precision.DEFAULT)
    x = num / wsum
    out = (jnp.dot(x, WxT_ref[...], precision=lax.Precision.DEFAULT)
           + jnp.dot(res_ref[...], WrT_ref[...], precision=lax.Precision.DEFAULT)
           + b_ref[...])
    out_ref[...] = out


@jax.jit
def _upsample(pos_pad, res, pos_cT, x_c, WxT, WrT, b2):
    grid = (N_F // BF,)
    return pl.pallas_call(
        _upsample_block,
        grid=grid,
        in_specs=[
            pl.BlockSpec((BF, 8), lambda i: (i, 0)),
            pl.BlockSpec((BF, D_RES), lambda i: (i, 0)),
            pl.BlockSpec((8, N_C), lambda i: (0, 0)),
            pl.BlockSpec((N_C, D_FEAT), lambda i: (0, 0)),
            pl.BlockSpec((D_FEAT, D_OUT), lambda i: (0, 0)),
            pl.BlockSpec((D_RES, D_OUT), lambda i: (0, 0)),
            pl.BlockSpec((1, D_OUT), lambda i: (0, 0)),
        ],
        out_specs=pl.BlockSpec((BF, D_OUT), lambda i: (i, 0)),
        out_shape=jax.ShapeDtypeStruct((N_F, D_OUT), jnp.float32),
    )(pos_pad, res, pos_cT, x_c, WxT, WrT, b2)


def kernel(x_c, pos_c, batch_c, res, pos, batch, W, b):
    pos_pad = jnp.pad(pos, ((0, 0), (0, 5)))
    pos_cT = jnp.pad(pos_c, ((0, 0), (0, 5))).T
    WxT = W[:, :D_FEAT].T
    WrT = W[:, D_FEAT:].T
    b2 = b.reshape(1, D_OUT)
    out = _upsample(pos_pad, res, pos_cT, x_c, WxT, WrT, b2)
    return (out, pos, batch)


# speedup vs baseline: 11.5732x; 1.0623x over previous
import functools

import jax
import jax.numpy as jnp
from jax import lax
from jax.experimental import pallas as pl

K = 3
N_C = 4096
N_F = 16384
D_FEAT = 256
D_RES = 256
D_OUT = 512
BF = 256
_BIG = 3.4e38


def _upsample_block(pos_ref, res_ref, pos_cT_ref, x_c_ref,
                    WxT_ref, WrT_ref, b_ref, iota_ref, out_ref):
    posb = pos_ref[...]
    pcT = pos_cT_ref[...]
    xy = jnp.dot(posb, pcT, precision=lax.Precision.DEFAULT)
    y2 = jnp.sum(posb * posb, axis=1, keepdims=True)
    x2 = jnp.sum(pcT * pcT, axis=0, keepdims=True)
    dist = (y2 + x2) - 2.0 * xy

    dex = ((posb[:, 0:1] - pcT[0:1, :]) ** 2
           + (posb[:, 1:2] - pcT[1:2, :]) ** 2
           + (posb[:, 2:3] - pcT[2:3, :]) ** 2)

    iota_row = iota_ref[...]
    for _ in range(K):
        m = jnp.min(dist, axis=1, keepdims=True)
        candf = jnp.where(dist == m, iota_row, _BIG)
        aminf = jnp.min(candf, axis=1, keepdims=True)
        dist = jnp.where(candf == aminf, _BIG, dist)

    sel = dist >= 3.0e38
    A = jnp.where(sel, 1.0 / jnp.maximum(dex, 1e-16), 0.0)
    wsum = jnp.sum(A, axis=1, keepdims=True)
    num = jnp.dot(A, x_c_ref[...], precision=lax.Precision.DEFAULT)
    x = num / wsum
    out = (jnp.dot(x, WxT_ref[...], precision=lax.Precision.DEFAULT)
           + jnp.dot(res_ref[...], WrT_ref[...], precision=lax.Precision.DEFAULT)
           + b_ref[...])
    out_ref[...] = out


@jax.jit
def _upsample(pos_pad, res, pos_cT, x_c, WxT, WrT, b2, iota_f):
    grid = (N_F // BF,)
    return pl.pallas_call(
        _upsample_block,
        grid=grid,
        in_specs=[
            pl.BlockSpec((BF, 8), lambda i: (i, 0)),
            pl.BlockSpec((BF, D_RES), lambda i: (i, 0)),
            pl.BlockSpec((8, N_C), lambda i: (0, 0)),
            pl.BlockSpec((N_C, D_FEAT), lambda i: (0, 0)),
            pl.BlockSpec((D_FEAT, D_OUT), lambda i: (0, 0)),
            pl.BlockSpec((D_RES, D_OUT), lambda i: (0, 0)),
            pl.BlockSpec((1, D_OUT), lambda i: (0, 0)),
            pl.BlockSpec((1, N_C), lambda i: (0, 0)),
        ],
        out_specs=pl.BlockSpec((BF, D_OUT), lambda i: (i, 0)),
        out_shape=jax.ShapeDtypeStruct((N_F, D_OUT), jnp.float32),
    )(pos_pad, res, pos_cT, x_c, WxT, WrT, b2, iota_f)


def kernel(x_c, pos_c, batch_c, res, pos, batch, W, b):
    pos_pad = jnp.pad(pos, ((0, 0), (0, 5)))
    pos_cT = jnp.pad(pos_c, ((0, 0), (0, 5))).T
    WxT = W[:, :D_FEAT].T
    WrT = W[:, D_FEAT:].T
    b2 = b.reshape(1, D_OUT)
    iota_f = jnp.arange(N_C, dtype=jnp.float32).reshape(1, N_C)
    out = _upsample(pos_pad, res, pos_cT, x_c, WxT, WrT, b2, iota_f)
    return (out, pos, batch)
